# baseline (speedup 1.0000x reference)
import functools

import jax
import jax.numpy as jnp
from jax import lax
from jax.experimental import pallas as pl
from jax.experimental.pallas import tpu as pltpu
from jax.experimental.pallas import tpu_sc as plsc

N = 10000
E = 160000
D = 256
NP = 10240
GARB = NP - 1
K = 128
NT = 1280
EP = NT * K
NSUB = 16
NCORE = 2
ROWS_PER_SUB = NP // NSUB
TILES_PER_SUB = NT // NSUB
TILES_PER_WORKER = NT // (NSUB * NCORE)
BN = 256
CHUNK = 8


@functools.cache
def _sc_kernels():
    mesh = plsc.VectorSubcoreMesh(core_axis_name="c", subcore_axis_name="s",
                                  num_cores=NCORE, num_subcores=NSUB)

    @functools.partial(
        pl.kernel,
        out_type=jax.ShapeDtypeStruct((NCORE * NP,), jnp.float32),
        mesh=mesh,
        scratch_types=[
            pltpu.VMEM((K,), jnp.int32),
            pltpu.VMEM((K,), jnp.float32),
            pltpu.VMEM_SHARED((NP,), jnp.float32),
        ],
    )
    def _sc_degree(dst_hbm, zeros1_hbm, out_hbm, dst_v, ones_v, acc):
        c = lax.axis_index("c")
        s = lax.axis_index("s")
        r0 = s * ROWS_PER_SUB
        pltpu.sync_copy(zeros1_hbm.at[pl.ds(r0, ROWS_PER_SUB)],
                        acc.at[pl.ds(r0, ROWS_PER_SUB)])

        @pl.loop(0, K, step=16)
        def _fill(j):
            ones_v[pl.ds(j, 16)] = jnp.ones((16,), jnp.float32)

        plsc.subcore_barrier()
        t0 = (c * NSUB + s) * TILES_PER_WORKER

        @pl.loop(0, TILES_PER_WORKER)
        def _body(i):
            pltpu.sync_copy(dst_hbm.at[t0 + i], dst_v)
            pltpu.sync_copy(ones_v, acc.at[dst_v], add=True)

        plsc.subcore_barrier()
        pltpu.sync_copy(acc.at[pl.ds(r0, ROWS_PER_SUB)],
                        out_hbm.at[pl.ds(c * NP + r0, ROWS_PER_SUB)])

    @functools.partial(
        pl.kernel,
        out_type=jax.ShapeDtypeStruct((NCORE * NP, 128), jnp.float32),
        mesh=mesh,
        scratch_types=[
            pltpu.VMEM((2 * CHUNK, K), jnp.int32),
            pltpu.VMEM((2 * CHUNK, K), jnp.int32),
            pltpu.VMEM((K, 128), jnp.float32),
            pltpu.VMEM((K, 128), jnp.float32),
            pltpu.VMEM_SHARED((NP, 128), jnp.float32),
            pltpu.SemaphoreType.DMA,
            pltpu.SemaphoreType.DMA,
        ],
    )
    def _sc_propagate(g_hbm, src2_hbm, dst_hbm, zeros_hbm, out_hbm,
                      sidx, didx, rows0, rows1, acc, g0, g1):
        c = lax.axis_index("c")
        s = lax.axis_index("s")
        r0 = s * ROWS_PER_SUB
        t0 = s * TILES_PER_SUB
        pltpu.sync_copy(src2_hbm.at[pl.ds(c * NT + t0, 2 * CHUNK)], sidx)
        pltpu.sync_copy(dst_hbm.at[pl.ds(t0, 2 * CHUNK)], didx)
        pltpu.sync_copy(zeros_hbm.at[pl.ds(r0, ROWS_PER_SUB)],
                        acc.at[pl.ds(r0, ROWS_PER_SUB)])
        plsc.subcore_barrier()

        def _gather_halves(ring_row, rows, sem):
            pltpu.async_copy(g_hbm.at[sidx.at[ring_row, pl.ds(0, 64)]],
                             rows.at[pl.ds(0, 64)], sem)
            pltpu.async_copy(g_hbm.at[sidx.at[ring_row, pl.ds(64, 64)]],
                             rows.at[pl.ds(64, 64)], sem)

        def _wait_halves(rows, sem):
            pltpu.make_async_copy(g_hbm.at[sidx.at[0, pl.ds(0, 64)]],
                                  rows.at[pl.ds(0, 64)], sem).wait()
            pltpu.make_async_copy(g_hbm.at[sidx.at[0, pl.ds(0, 64)]],
                                  rows.at[pl.ds(64, 64)], sem).wait()

        _gather_halves(0, rows0, g0)
        _gather_halves(1, rows1, g1)
        nchunks = TILES_PER_SUB // CHUNK

        @pl.loop(0, nchunks)
        def _chunk(ci):
            half = lax.rem(ci, 2) * CHUNK

            @pl.when(jnp.logical_and(ci >= 1, ci < nchunks - 1))
            def _():
                other = CHUNK - half
                pltpu.sync_copy(
                    src2_hbm.at[pl.ds(c * NT + t0 + (ci + 1) * CHUNK, CHUNK)],
                    sidx.at[pl.ds(other, CHUNK)])
                pltpu.sync_copy(
                    dst_hbm.at[pl.ds(t0 + (ci + 1) * CHUNK, CHUNK)],
                    didx.at[pl.ds(other, CHUNK)])

            for j in range(CHUNK):
                t = ci * CHUNK + j
                rows, sem = (rows0, g0) if j % 2 == 0 else (rows1, g1)
                _wait_halves(rows, sem)
                pltpu.sync_copy(rows, acc.at[didx.at[half + j]], add=True)

                @pl.when(t + 2 < TILES_PER_SUB)
                def _():
                    nxt = lax.rem(t + 2, 2 * CHUNK)
                    _gather_halves(nxt, rows, sem)

        plsc.subcore_barrier()
        pltpu.sync_copy(acc.at[pl.ds(r0, ROWS_PER_SUB)],
                        out_hbm.at[pl.ds(c * NP + r0, ROWS_PER_SUB)])

    return _sc_degree, _sc_propagate


def _mm_body(x_ref, w_ref, o_ref):
    m = jnp.dot(x_ref[...], w_ref[...], preferred_element_type=jnp.float32)
    o_ref[0] = m[:, :128]
    o_ref[1] = m[:, 128:]


def _tc_matmul_split(x, w):
    return pl.pallas_call(
        _mm_body,
        grid=(NP // BN,),
        in_specs=[pl.BlockSpec((BN, D), lambda i: (i, 0)),
                  pl.BlockSpec((D, D), lambda i: (0, 0))],
        out_specs=pl.BlockSpec((NCORE, BN, 128), lambda i: (0, i, 0)),
        out_shape=jax.ShapeDtypeStruct((NCORE, NP, 128), jnp.float32),
    )(x, w)


def _scale_body(d_ref, m_ref, g_ref, dv_ref):
    dv = lax.rsqrt(1.0 + d_ref[0] + d_ref[1])
    dvw = jnp.broadcast_to(dv, (BN, 128))
    dv_ref[...] = dvw
    g_ref[0] = dvw * m_ref[0]
    g_ref[1] = dvw * m_ref[1]


def _tc_scale(deg2, m1):
    return pl.pallas_call(
        _scale_body,
        grid=(NP // BN,),
        in_specs=[pl.BlockSpec((NCORE, BN, 1), lambda i: (0, i, 0)),
                  pl.BlockSpec((NCORE, BN, 128), lambda i: (0, i, 0))],
        out_specs=[pl.BlockSpec((NCORE, BN, 128), lambda i: (0, i, 0)),
                   pl.BlockSpec((BN, 128), lambda i: (i, 0))],
        out_shape=[jax.ShapeDtypeStruct((NCORE, NP, 128), jnp.float32),
                   jax.ShapeDtypeStruct((NP, 128), jnp.float32)],
    )(deg2, m1)


def _fused_body(s_ref, g_ref, dv_ref, b_ref, w_ref, o_ref):
    dv = dv_ref[...]
    b = b_ref[...]
    y0 = jnp.maximum(dv * (s_ref[0] + g_ref[0]) + b[:, :128], 0.0)
    y1 = jnp.maximum(dv * (s_ref[1] + g_ref[1]) + b[:, 128:], 0.0)
    y = jnp.concatenate([y0, y1], axis=1)
    m = jnp.dot(y, w_ref[...], preferred_element_type=jnp.float32)
    o_ref[0] = dv * m[:, :128]
    o_ref[1] = dv * m[:, 128:]


def _tc_fused(S, g, dinvw, b, w):
    return pl.pallas_call(
        _fused_body,
        grid=(NP // BN,),
        in_specs=[pl.BlockSpec((NCORE, BN, 128), lambda i: (0, i, 0)),
                  pl.BlockSpec((NCORE, BN, 128), lambda i: (0, i, 0)),
                  pl.BlockSpec((BN, 128), lambda i: (i, 0)),
                  pl.BlockSpec((1, D), lambda i: (0, 0)),
                  pl.BlockSpec((D, D), lambda i: (0, 0))],
        out_specs=pl.BlockSpec((NCORE, BN, 128), lambda i: (0, i, 0)),
        out_shape=jax.ShapeDtypeStruct((NCORE, NP, 128), jnp.float32),
    )(S, g, dinvw, b, w)


def _epilogue_body(s_ref, g_ref, dv_ref, b_ref, o_ref):
    dv = dv_ref[...]
    b = b_ref[...]
    y0 = dv * (s_ref[0] + g_ref[0]) + b[:, :128]
    y1 = dv * (s_ref[1] + g_ref[1]) + b[:, 128:]
    o_ref[...] = jnp.concatenate([y0, y1], axis=1)


def _tc_epilogue(S, g, dinvw, b):
    return pl.pallas_call(
        _epilogue_body,
        grid=(NP // BN,),
        in_specs=[pl.BlockSpec((NCORE, BN, 128), lambda i: (0, i, 0)),
                  pl.BlockSpec((NCORE, BN, 128), lambda i: (0, i, 0)),
                  pl.BlockSpec((BN, 128), lambda i: (i, 0)),
                  pl.BlockSpec((1, D), lambda i: (0, 0))],
        out_specs=pl.BlockSpec((BN, D), lambda i: (i, 0)),
        out_shape=jax.ShapeDtypeStruct((NP, D), jnp.float32),
    )(S, g, dinvw, b)


def kernel(x, edge_index, W1, b1, W2, b2, W3, b3):
    src = edge_index[0]
    dst = edge_index[1]
    pad = EP - E
    srcp = jnp.concatenate([src, jnp.zeros((pad,), jnp.int32)])
    dstp = jnp.concatenate([dst, jnp.full((pad,), GARB, jnp.int32)])
    src2 = jnp.concatenate([srcp, srcp + NP]).reshape(NCORE * NT, K)
    dstt = dstp.reshape(NT, K)
    x_pad = jnp.pad(x, ((0, NP - N), (0, 0)))
    zeros1 = jnp.zeros((NP,), jnp.float32)
    zeros2 = jnp.zeros((NP, 128), jnp.float32)
    b1r = b1.reshape(1, D)
    b2r = b2.reshape(1, D)
    b3r = b3.reshape(1, D)

    _sc_degree, _sc_propagate = _sc_kernels()
    deg2 = _sc_degree(dstt, zeros1)
    m1 = _tc_matmul_split(x_pad, W1)
    g1, dinvw = _tc_scale(deg2.reshape(NCORE, NP, 1), m1)

    g1f = g1.reshape(NCORE * NP, 128)
    S1 = _sc_propagate(g1f, src2, dstt, zeros2)
    g2 = _tc_fused(S1.reshape(NCORE, NP, 128), g1, dinvw, b1r, W2)

    g2f = g2.reshape(NCORE * NP, 128)
    S2 = _sc_propagate(g2f, src2, dstt, zeros2)
    g3 = _tc_fused(S2.reshape(NCORE, NP, 128), g2, dinvw, b2r, W3)

    g3f = g3.reshape(NCORE * NP, 128)
    S3 = _sc_propagate(g3f, src2, dstt, zeros2)
    out = _tc_epilogue(S3.reshape(NCORE, NP, 128), g3, dinvw, b3r)
    return out[:N]


# speedup vs baseline: 1.0005x; 1.0005x over previous
import functools

import jax
import jax.numpy as jnp
from jax import lax
from jax.experimental import pallas as pl
from jax.experimental.pallas import tpu as pltpu
from jax.experimental.pallas import tpu_sc as plsc

N = 10000
E = 160000
D = 256
NP = 10240
GARB = NP - 1
K = 128
NT = 1280
EP = NT * K
NSUB = 16
NCORE = 2
ROWS_PER_SUB = NP // NSUB
TILES_PER_SUB = NT // NSUB
TILES_PER_WORKER = NT // (NSUB * NCORE)
BN = 256
CHUNK = 8


@functools.cache
def _sc_kernels():
    mesh = plsc.VectorSubcoreMesh(core_axis_name="c", subcore_axis_name="s",
                                  num_cores=NCORE, num_subcores=NSUB)

    @functools.partial(
        pl.kernel,
        out_type=jax.ShapeDtypeStruct((NCORE * NP,), jnp.float32),
        mesh=mesh,
        scratch_types=[
            pltpu.VMEM((K,), jnp.int32),
            pltpu.VMEM((K,), jnp.float32),
            pltpu.VMEM_SHARED((NP,), jnp.float32),
        ],
    )
    def _sc_degree(dst_hbm, zeros1_hbm, out_hbm, dst_v, ones_v, acc):
        c = lax.axis_index("c")
        s = lax.axis_index("s")
        r0 = s * ROWS_PER_SUB
        pltpu.sync_copy(zeros1_hbm.at[pl.ds(r0, ROWS_PER_SUB)],
                        acc.at[pl.ds(r0, ROWS_PER_SUB)])

        @pl.loop(0, K, step=16)
        def _fill(j):
            ones_v[pl.ds(j, 16)] = jnp.ones((16,), jnp.float32)

        plsc.subcore_barrier()
        t0 = (c * NSUB + s) * TILES_PER_WORKER

        @pl.loop(0, TILES_PER_WORKER)
        def _body(i):
            pltpu.sync_copy(dst_hbm.at[t0 + i], dst_v)
            pltpu.sync_copy(ones_v, acc.at[dst_v], add=True)

        plsc.subcore_barrier()
        pltpu.sync_copy(acc.at[pl.ds(r0, ROWS_PER_SUB)],
                        out_hbm.at[pl.ds(c * NP + r0, ROWS_PER_SUB)])

    @functools.partial(
        pl.kernel,
        out_type=jax.ShapeDtypeStruct((NCORE * NP, 128), jnp.float32),
        mesh=mesh,
        scratch_types=[
            pltpu.VMEM((2 * CHUNK, K), jnp.int32),
            pltpu.VMEM((2 * CHUNK, K), jnp.int32),
            pltpu.VMEM((K, 128), jnp.float32),
            pltpu.VMEM((K, 128), jnp.float32),
            pltpu.VMEM_SHARED((NP, 128), jnp.float32),
            pltpu.SemaphoreType.DMA,
            pltpu.SemaphoreType.DMA,
        ],
    )
    def _sc_propagate(g_hbm, src2_hbm, dst_hbm, zeros_hbm, out_hbm,
                      sidx, didx, rows0, rows1, acc, g0, g1):
        c = lax.axis_index("c")
        s = lax.axis_index("s")
        r0 = s * ROWS_PER_SUB
        t0 = s * TILES_PER_SUB
        pltpu.sync_copy(src2_hbm.at[pl.ds(c * NT + t0, 2 * CHUNK)], sidx)
        pltpu.sync_copy(dst_hbm.at[pl.ds(t0, 2 * CHUNK)], didx)
        pltpu.sync_copy(zeros_hbm.at[pl.ds(r0, ROWS_PER_SUB)],
                        acc.at[pl.ds(r0, ROWS_PER_SUB)])
        plsc.subcore_barrier()

        pltpu.async_copy(g_hbm.at[sidx.at[0]], rows0, g0)
        pltpu.async_copy(g_hbm.at[sidx.at[1]], rows1, g1)
        nchunks = TILES_PER_SUB // CHUNK

        @pl.loop(0, nchunks)
        def _chunk(ci):
            half = lax.rem(ci, 2) * CHUNK

            @pl.when(jnp.logical_and(ci >= 1, ci < nchunks - 1))
            def _():
                other = CHUNK - half
                pltpu.sync_copy(
                    src2_hbm.at[pl.ds(c * NT + t0 + (ci + 1) * CHUNK, CHUNK)],
                    sidx.at[pl.ds(other, CHUNK)])
                pltpu.sync_copy(
                    dst_hbm.at[pl.ds(t0 + (ci + 1) * CHUNK, CHUNK)],
                    didx.at[pl.ds(other, CHUNK)])

            for j in range(CHUNK):
                t = ci * CHUNK + j
                rows, sem = (rows0, g0) if j % 2 == 0 else (rows1, g1)
                pltpu.make_async_copy(g_hbm.at[sidx.at[0]], rows, sem).wait()
                pltpu.sync_copy(rows, acc.at[didx.at[half + j]], add=True)

                @pl.when(t + 2 < TILES_PER_SUB)
                def _():
                    nxt = lax.rem(t + 2, 2 * CHUNK)
                    pltpu.async_copy(g_hbm.at[sidx.at[nxt]], rows, sem)

        plsc.subcore_barrier()
        pltpu.sync_copy(acc.at[pl.ds(r0, ROWS_PER_SUB)],
                        out_hbm.at[pl.ds(c * NP + r0, ROWS_PER_SUB)])

    return _sc_degree, _sc_propagate


def _mm_body(x_ref, w_ref, o_ref):
    m = jnp.dot(x_ref[...], w_ref[...], preferred_element_type=jnp.float32)
    o_ref[0] = m[:, :128]
    o_ref[1] = m[:, 128:]


def _tc_matmul_split(x, w):
    return pl.pallas_call(
        _mm_body,
        grid=(NP // BN,),
        in_specs=[pl.BlockSpec((BN, D), lambda i: (i, 0)),
                  pl.BlockSpec((D, D), lambda i: (0, 0))],
        out_specs=pl.BlockSpec((NCORE, BN, 128), lambda i: (0, i, 0)),
        out_shape=jax.ShapeDtypeStruct((NCORE, NP, 128), jnp.float32),
    )(x, w)


def _scale_body(d_ref, m_ref, g_ref, dv_ref):
    dv = lax.rsqrt(1.0 + d_ref[0] + d_ref[1])
    dvw = jnp.broadcast_to(dv, (BN, 128))
    dv_ref[...] = dvw
    g_ref[0] = dvw * m_ref[0]
    g_ref[1] = dvw * m_ref[1]


def _tc_scale(deg2, m1):
    return pl.pallas_call(
        _scale_body,
        grid=(NP // BN,),
        in_specs=[pl.BlockSpec((NCORE, BN, 1), lambda i: (0, i, 0)),
                  pl.BlockSpec((NCORE, BN, 128), lambda i: (0, i, 0))],
        out_specs=[pl.BlockSpec((NCORE, BN, 128), lambda i: (0, i, 0)),
                   pl.BlockSpec((BN, 128), lambda i: (i, 0))],
        out_shape=[jax.ShapeDtypeStruct((NCORE, NP, 128), jnp.float32),
                   jax.ShapeDtypeStruct((NP, 128), jnp.float32)],
    )(deg2, m1)


def _fused_body(s_ref, g_ref, dv_ref, b_ref, w_ref, o_ref):
    dv = dv_ref[...]
    b = b_ref[...]
    y0 = jnp.maximum(dv * (s_ref[0] + g_ref[0]) + b[:, :128], 0.0)
    y1 = jnp.maximum(dv * (s_ref[1] + g_ref[1]) + b[:, 128:], 0.0)
    y = jnp.concatenate([y0, y1], axis=1)
    m = jnp.dot(y, w_ref[...], preferred_element_type=jnp.float32)
    o_ref[0] = dv * m[:, :128]
    o_ref[1] = dv * m[:, 128:]


def _tc_fused(S, g, dinvw, b, w):
    return pl.pallas_call(
        _fused_body,
        grid=(NP // BN,),
        in_specs=[pl.BlockSpec((NCORE, BN, 128), lambda i: (0, i, 0)),
                  pl.BlockSpec((NCORE, BN, 128), lambda i: (0, i, 0)),
                  pl.BlockSpec((BN, 128), lambda i: (i, 0)),
                  pl.BlockSpec((1, D), lambda i: (0, 0)),
                  pl.BlockSpec((D, D), lambda i: (0, 0))],
        out_specs=pl.BlockSpec((NCORE, BN, 128), lambda i: (0, i, 0)),
        out_shape=jax.ShapeDtypeStruct((NCORE, NP, 128), jnp.float32),
    )(S, g, dinvw, b, w)


def _epilogue_body(s_ref, g_ref, dv_ref, b_ref, o_ref):
    dv = dv_ref[...]
    b = b_ref[...]
    y0 = dv * (s_ref[0] + g_ref[0]) + b[:, :128]
    y1 = dv * (s_ref[1] + g_ref[1]) + b[:, 128:]
    o_ref[...] = jnp.concatenate([y0, y1], axis=1)


def _tc_epilogue(S, g, dinvw, b):
    return pl.pallas_call(
        _epilogue_body,
        grid=(NP // BN,),
        in_specs=[pl.BlockSpec((NCORE, BN, 128), lambda i: (0, i, 0)),
                  pl.BlockSpec((NCORE, BN, 128), lambda i: (0, i, 0)),
                  pl.BlockSpec((BN, 128), lambda i: (i, 0)),
                  pl.BlockSpec((1, D), lambda i: (0, 0))],
        out_specs=pl.BlockSpec((BN, D), lambda i: (i, 0)),
        out_shape=jax.ShapeDtypeStruct((NP, D), jnp.float32),
    )(S, g, dinvw, b)


def kernel(x, edge_index, W1, b1, W2, b2, W3, b3):
    src = edge_index[0]
    dst = edge_index[1]
    pad = EP - E
    srcp = jnp.concatenate([src, jnp.zeros((pad,), jnp.int32)])
    dstp = jnp.concatenate([dst, jnp.full((pad,), GARB, jnp.int32)])
    src2 = jnp.concatenate([srcp, srcp + NP]).reshape(NCORE * NT, K)
    dstt = dstp.reshape(NT, K)
    x_pad = jnp.pad(x, ((0, NP - N), (0, 0)))
    zeros1 = jnp.zeros((NP,), jnp.float32)
    zeros2 = jnp.zeros((NP, 128), jnp.float32)
    b1r = b1.reshape(1, D)
    b2r = b2.reshape(1, D)
    b3r = b3.reshape(1, D)

    _sc_degree, _sc_propagate = _sc_kernels()
    deg2 = _sc_degree(dstt, zeros1)
    m1 = _tc_matmul_split(x_pad, W1)
    g1, dinvw = _tc_scale(deg2.reshape(NCORE, NP, 1), m1)

    g1f = g1.reshape(NCORE * NP, 128)
    S1 = _sc_propagate(g1f, src2, dstt, zeros2)
    g2 = _tc_fused(S1.reshape(NCORE, NP, 128), g1, dinvw, b1r, W2)

    g2f = g2.reshape(NCORE * NP, 128)
    S2 = _sc_propagate(g2f, src2, dstt, zeros2)
    g3 = _tc_fused(S2.reshape(NCORE, NP, 128), g2, dinvw, b2r, W3)

    g3f = g3.reshape(NCORE * NP, 128)
    S3 = _sc_propagate(g3f, src2, dstt, zeros2)
    out = _tc_epilogue(S3.reshape(NCORE, NP, 128), g3, dinvw, b3r)
    return out[:N]
